# Initial kernel scaffold; baseline (speedup 1.0000x reference)
import jax
import jax.numpy as jnp
from jax.experimental import pallas as pl


def kernel(H, neighbor_msg, W_score, theta):
    raise NotImplementedError("write your pallas kernel here")


# speedup vs baseline: 60.6150x; 60.6150x over previous
import jax
import jax.numpy as jnp
from jax.experimental import pallas as pl
from jax.experimental.pallas import tpu as pltpu

_R, _B, _D = 256, 32, 1024
_K = 6
_RBLK = 32
_NBLK = _R // _RBLK


def _adj_body(h_ref, nm_ref, w_ref, th_ref, adj_ref):
    h = h_ref[...]
    nm = nm_ref[...]
    w = w_ref[...]
    s = jnp.dot(h.reshape(_RBLK * _B, _D), w,
                preferred_element_type=jnp.float32).reshape(_RBLK, _B)
    fb = jnp.sqrt(jnp.sum(nm * nm, axis=-1))
    th = th_ref[...]
    adj_ref[...] = s - th - 0.5 * ((1.0 - 0.9) * fb)


def _nms_body(adj_ref, hard_ref):
    adj = adj_ref[...]
    iota = jax.lax.broadcasted_iota(jnp.int32, (_B, _R), 1)

    mask = jnp.zeros((_B, _R), jnp.float32)
    sup = jnp.zeros((_B, _R), jnp.float32)
    for _ in range(_K):
        cur = jnp.where(sup > 0, -jnp.inf, adj)
        m = jnp.max(cur, axis=1, keepdims=True)
        cand = jnp.where(cur == m, iota, _R)
        idx = jnp.min(cand, axis=1, keepdims=True)
        sel = iota == idx
        sel_r = iota == ((idx + 1) % _R)
        sel_l = iota == ((idx + _R - 1) % _R)
        mask = jnp.where(sel, 1.0, mask)
        sup = jnp.where(sel | sel_l | sel_r, 1.0, sup)
    hard_ref[...] = mask


def _scale_body(h_ref, m_ref, out_ref):
    out_ref[...] = h_ref[...] * m_ref[...][:, :, None]


def kernel(H, neighbor_msg, W_score, theta):
    adj_t = pl.pallas_call(
        _adj_body,
        grid=(_NBLK,),
        in_specs=[
            pl.BlockSpec((_RBLK, _B, _D), lambda i: (i, 0, 0)),
            pl.BlockSpec((_RBLK, _B, _D), lambda i: (i, 0, 0)),
            pl.BlockSpec((_D, 1), lambda i: (0, 0)),
            pl.BlockSpec((_RBLK, 1), lambda i: (i, 0)),
        ],
        out_specs=pl.BlockSpec((_RBLK, _B), lambda i: (i, 0)),
        out_shape=jax.ShapeDtypeStruct((_R, _B), jnp.float32),
    )(H, neighbor_msg, W_score, theta.reshape(_R, 1))

    adj = adj_t.T

    hard = pl.pallas_call(
        _nms_body,
        out_shape=jax.ShapeDtypeStruct((_B, _R), jnp.float32),
    )(adj)

    Hs = pl.pallas_call(
        _scale_body,
        grid=(_NBLK,),
        in_specs=[
            pl.BlockSpec((_RBLK, _B, _D), lambda i: (i, 0, 0)),
            pl.BlockSpec((_RBLK, _B), lambda i: (i, 0)),
        ],
        out_specs=pl.BlockSpec((_RBLK, _B, _D), lambda i: (i, 0, 0)),
        out_shape=jax.ShapeDtypeStruct((_R, _B, _D), jnp.float32),
    )(H, hard.T)

    return (Hs, hard, adj)
